# baseline (speedup 1.0000x reference)
import functools

import jax
import jax.numpy as jnp
from jax import lax
from jax.experimental import pallas as pl
from jax.experimental.pallas import tpu as pltpu
from jax.experimental.pallas import tpu_sc as plsc

N = 50000
D = 2048
NSUB = 5000
NSUB_PAD = 5120
B = 128
K = 30
NHID = 512
NCLASS = 2

DC = 256
NCHUNK = D // DC
NCORE = 2
NTILE = 16
ROWS_PER_TILE = N // NTILE
BLK = 125
NBLK = ROWS_PER_TILE // BLK
ACC_ROWS_PER_TILE = NSUB_PAD // NTILE


def _seg_sum_body(feat2, lab3, zfeat, zcnt16, out3, cnt, acc, cacc, fbuf, labv, idxv, onesv):
    c = lax.axis_index("c")
    s = lax.axis_index("s")
    ov = jnp.ones((16,), jnp.float32)
    iv = lax.iota(jnp.int32, 16)

    @pl.loop(0, BLK)
    def _(i):
        onesv[i, :] = ov

    pltpu.sync_copy(lab3.at[s], labv)
    r0 = s * ACC_ROWS_PER_TILE
    row0 = s * ROWS_PER_TILE

    for p in range(NCHUNK // NCORE):
        dc = p * NCORE + c
        @pl.loop(0, NBLK)
        def _(b):
            for g in range(8):
                r = jnp.minimum(row0 + b * BLK + g * 16 + iv, N - 1)
                idxv[b, pl.ds(g * 16, 16)] = r * NCHUNK + dc
        pltpu.sync_copy(zfeat, acc.at[pl.ds(r0, ACC_ROWS_PER_TILE)])
        if p == 0:
            @pl.when(c == 0)
            def _():
                pltpu.sync_copy(zcnt16, cacc.at[pl.ds(r0, ACC_ROWS_PER_TILE)])
        plsc.subcore_barrier()
        for b in range(NBLK):
            pltpu.sync_copy(feat2.at[idxv.at[b]], fbuf)
            pltpu.sync_copy(fbuf.at[pl.ds(0, BLK)], acc.at[labv.at[b]], add=True)
            if p == 0:
                @pl.when(c == 0)
                def _():
                    pltpu.sync_copy(onesv, cacc.at[labv.at[b]], add=True)
        plsc.subcore_barrier()
        pltpu.sync_copy(acc.at[pl.ds(r0, ACC_ROWS_PER_TILE)],
                        out3.at[dc, pl.ds(r0, ACC_ROWS_PER_TILE)])
        if p == 0:
            @pl.when(c == 0)
            def _():
                pltpu.sync_copy(cacc.at[pl.ds(r0, ACC_ROWS_PER_TILE)],
                                cnt.at[pl.ds(r0, ACC_ROWS_PER_TILE)])
        plsc.subcore_barrier()


@jax.jit
def _seg_sum_sc(feat2, lab3, zfeat, zcnt16):
    mesh = plsc.VectorSubcoreMesh(core_axis_name="c", subcore_axis_name="s")
    return pl.kernel(
        _seg_sum_body,
        out_type=(
            jax.ShapeDtypeStruct((NCHUNK, NSUB_PAD, DC), jnp.float32),
            jax.ShapeDtypeStruct((NSUB_PAD, 16), jnp.float32),
        ),
        mesh=mesh,
        compiler_params=pltpu.CompilerParams(use_tc_tiling_on_sc=False),
        scratch_types=[
            pltpu.VMEM_SHARED((NSUB_PAD, DC), jnp.float32),
            pltpu.VMEM_SHARED((NSUB_PAD, 16), jnp.float32),
            pltpu.VMEM((128, DC), jnp.float32),
            pltpu.VMEM((NBLK, BLK), jnp.int32),
            pltpu.VMEM((NBLK, 128), jnp.int32),
            pltpu.VMEM((BLK, 16), jnp.float32),
        ],
    )(feat2, lab3, zfeat, zcnt16)


def kernel(indexes, features, labels, sub_label, domain, ori_0, ori_knn_neighbor,
           all_pred, output_feat, conv_w, conv_b, fc1_w, fc1_b, prelu_w, fc2_w, fc2_b):
    feat2 = features.reshape(N * NCHUNK, DC)
    lab3 = sub_label.reshape(NTILE, NBLK, BLK)
    zfeat = jnp.zeros((ACC_ROWS_PER_TILE, DC), jnp.float32)
    zcnt16 = jnp.zeros((ACC_ROWS_PER_TILE, 16), jnp.float32)
    sums3, cnt = _seg_sum_sc(feat2, lab3, zfeat, zcnt16)
    sums = sums3.transpose(1, 0, 2).reshape(NSUB_PAD, D)[:NSUB]
    nums = cnt[:NSUB, :1]
    mask = (nums > 0).astype(jnp.float32)
    sub_sum = sums / (mask * nums + (1.0 - mask))

    def build(idx, nbrs):
        sub_lab = sub_label[nbrs]
        sub_feat = sub_sum[sub_lab]
        sub_feat = sub_feat.at[0].set(features[idx])
        A = (sub_feat @ sub_feat.T) / 5.0
        _, topi = jax.lax.top_k(A, 5)
        m = jnp.zeros_like(A).at[jnp.arange(A.shape[0])[:, None], topi].set(1.0)
        m = ((m > 0) & (m.T > 0)).astype(jnp.float32)
        A = A * m
        sub_feat = sub_feat / jnp.linalg.norm(sub_feat, axis=1, keepdims=True)
        sub_feat = sub_feat - sub_feat[0]
        return sub_feat, A

    all_x, all_adj = jax.vmap(build)(indexes, ori_knn_neighbor)
    agg = jnp.einsum('bij,bjd->bid', all_adj, all_x)
    cat = jnp.concatenate([all_x, agg], axis=2)
    h = jax.nn.relu(jnp.einsum('bnd,df->bnf', cat, conv_w) + conv_b)
    x0 = h.reshape(-1, NHID)
    z = x0 @ fc1_w + fc1_b
    z = jnp.where(z >= 0, z, prelu_w * z)
    logits = z @ fc2_w + fc2_b
    simm = jnp.sum(sub_sum * sub_sum, axis=1)
    pred = jax.nn.softmax(logits.reshape(indexes.shape[0], K, NCLASS), axis=2)
    return pred, simm, sub_sum, nums


# speedup vs baseline: 1.0826x; 1.0826x over previous
import functools

import jax
import jax.numpy as jnp
from jax import lax
from jax.experimental import pallas as pl
from jax.experimental.pallas import tpu as pltpu
from jax.experimental.pallas import tpu_sc as plsc

N = 50000
D = 2048
NSUB = 5000
NSUB_PAD = 5120
B = 128
K = 30
KPAD = 32
NHID = 512
NCLASS = 2

DC = 256
NCHUNK = D // DC
NCORE = 2
NTILE = 16
ROWS_PER_TILE = N // NTILE
BLK = 125
NBLK = ROWS_PER_TILE // BLK
ACC_ROWS_PER_TILE = NSUB_PAD // NTILE


def _seg_sum_body(feat2, lab3, zfeat, zcnt16, out3, cnt,
                  acc, cacc, fbuf, labv, idxv, onesv):
    c = lax.axis_index("c")
    s = lax.axis_index("s")
    ov = jnp.ones((16,), jnp.float32)
    iv = lax.iota(jnp.int32, 16)

    @pl.loop(0, BLK)
    def _(i):
        onesv[i, :] = ov

    pltpu.sync_copy(lab3.at[s], labv)
    r0 = s * ACC_ROWS_PER_TILE
    row0 = s * ROWS_PER_TILE

    for p in range(NCHUNK // NCORE):
        dc = p * NCORE + c
        @pl.loop(0, NBLK)
        def _(b):
            for g in range(8):
                r = jnp.minimum(row0 + b * BLK + g * 16 + iv, N - 1)
                idxv[b, pl.ds(g * 16, 16)] = r * NCHUNK + dc
        pltpu.sync_copy(zfeat, acc.at[pl.ds(r0, ACC_ROWS_PER_TILE)])
        if p == 0:
            @pl.when(c == 0)
            def _():
                pltpu.sync_copy(zcnt16, cacc.at[pl.ds(r0, ACC_ROWS_PER_TILE)])
        plsc.subcore_barrier()
        for b in range(NBLK):
            pltpu.sync_copy(feat2.at[idxv.at[b]], fbuf)
            pltpu.sync_copy(fbuf.at[pl.ds(0, BLK)], acc.at[labv.at[b]], add=True)
            if p == 0:
                @pl.when(c == 0)
                def _():
                    pltpu.sync_copy(onesv, cacc.at[labv.at[b]], add=True)
        plsc.subcore_barrier()
        pltpu.sync_copy(acc.at[pl.ds(r0, ACC_ROWS_PER_TILE)],
                        out3.at[dc, pl.ds(r0, ACC_ROWS_PER_TILE)])
        if p == 0:
            @pl.when(c == 0)
            def _():
                pltpu.sync_copy(cacc.at[pl.ds(r0, ACC_ROWS_PER_TILE)],
                                cnt.at[pl.ds(r0, ACC_ROWS_PER_TILE)])
        plsc.subcore_barrier()


def _seg_sum_sc(feat2, lab3, zfeat, zcnt16):
    mesh = plsc.VectorSubcoreMesh(core_axis_name="c", subcore_axis_name="s")
    return pl.kernel(
        _seg_sum_body,
        out_type=(
            jax.ShapeDtypeStruct((NCHUNK, NSUB_PAD, DC), jnp.float32),
            jax.ShapeDtypeStruct((NSUB_PAD, 16), jnp.float32),
        ),
        mesh=mesh,
        compiler_params=pltpu.CompilerParams(use_tc_tiling_on_sc=False),
        scratch_types=[
            pltpu.VMEM_SHARED((NSUB_PAD, DC), jnp.float32),
            pltpu.VMEM_SHARED((NSUB_PAD, 16), jnp.float32),
            pltpu.VMEM((128, DC), jnp.float32),
            pltpu.VMEM((NBLK, BLK), jnp.int32),
            pltpu.VMEM((NBLK, 128), jnp.int32),
            pltpu.VMEM((BLK, 16), jnp.float32),
        ],
    )(feat2, lab3, zfeat, zcnt16)


NORM_ROWS = 200


def _norm_body(sums_ref, cnt_ref, sub_ref, simm_ref, nums_ref):
    c = cnt_ref[:, :1]
    scale = jnp.where(c > 0, 1.0 / jnp.where(c > 0, c, 1.0), 1.0)
    for q in range(NCHUNK):
        sub_ref[:, q * DC:(q + 1) * DC] = sums_ref[q] * scale
    w = sub_ref[...]
    simm_ref[...] = jnp.sum(w * w, axis=1, keepdims=True)
    nums_ref[...] = c


def _normalize_tc(sums3, cnt):
    grid = NSUB // NORM_ROWS
    return pl.pallas_call(
        _norm_body,
        grid=(grid,),
        in_specs=[
            pl.BlockSpec((NCHUNK, NORM_ROWS, DC), lambda i: (0, i, 0)),
            pl.BlockSpec((NORM_ROWS, 16), lambda i: (i, 0)),
        ],
        out_specs=[
            pl.BlockSpec((NORM_ROWS, D), lambda i: (i, 0)),
            pl.BlockSpec((NORM_ROWS, 1), lambda i: (i, 0)),
            pl.BlockSpec((NORM_ROWS, 1), lambda i: (i, 0)),
        ],
        out_shape=[
            jax.ShapeDtypeStruct((NSUB, D), jnp.float32),
            jax.ShapeDtypeStruct((NSUB, 1), jnp.float32),
            jax.ShapeDtypeStruct((NSUB, 1), jnp.float32),
        ],
    )(sums3, cnt)


AB = 4
NEG = -1e30


def _gcn_body(labs_ref, idxs_ref, sub_hbm, feat_hbm,
              conv_w_ref, conv_b_ref, fc1_w_ref, fc1_b_ref,
              prelu_ref, fc2_w_ref, fc2_b_ref,
              pred_ref, xbuf, sem):
    copies = []
    for a in range(AB):
        copies.append(pltpu.make_async_copy(
            feat_hbm.at[pl.ds(idxs_ref[0, 0, a], 1)], xbuf.at[a, pl.ds(0, 1)], sem))
        copies[-1].start()
        for j in range(1, K):
            copies.append(pltpu.make_async_copy(
                sub_hbm.at[pl.ds(labs_ref[0, a, j], 1)], xbuf.at[a, pl.ds(j, 1)], sem))
            copies[-1].start()
    for cp in copies:
        cp.wait()

    rows = lax.broadcasted_iota(jnp.int32, (KPAD, KPAD), 0)
    cols = lax.broadcasted_iota(jnp.int32, (KPAD, KPAD), 1)
    colpad = cols >= K

    for a in range(AB):
        xbuf[a, pl.ds(K, KPAD - K)] = jnp.zeros((KPAD - K, D), jnp.float32)
        X = xbuf[a]
        A = lax.dot_general(X, X, (((1,), (1,)), ((), ())),
                            preferred_element_type=jnp.float32) * 0.2
        cur = jnp.where(colpad, NEG, A)
        msk = jnp.zeros((KPAD, KPAD), jnp.float32)
        for _ in range(5):
            mx = jnp.max(cur, axis=1, keepdims=True)
            cand = jnp.where(cur == mx, cols, KPAD)
            cmin = jnp.min(cand, axis=1, keepdims=True)
            sel = cols == cmin
            msk = jnp.where(sel, 1.0, msk)
            cur = jnp.where(sel, NEG, cur)
        eye = (rows == cols).astype(jnp.float32)
        mskT = lax.dot_general(msk, eye, (((0,), (0,)), ((), ())),
                               preferred_element_type=jnp.float32)
        Am = A * msk * jnp.where(mskT > 0, 1.0, 0.0)
        n2 = jnp.sum(X * X, axis=1, keepdims=True)
        rn = jnp.where(n2 > 0, lax.rsqrt(jnp.where(n2 > 0, n2, 1.0)), 0.0)
        Xn = X * rn
        Xn = Xn - Xn[0:1, :]
        agg = lax.dot_general(Am, Xn, (((1,), (0,)), ((), ())),
                              preferred_element_type=jnp.float32)
        h = (lax.dot_general(Xn, conv_w_ref[pl.ds(0, D)], (((1,), (0,)), ((), ())),
                             preferred_element_type=jnp.float32)
             + lax.dot_general(agg, conv_w_ref[pl.ds(D, D)], (((1,), (0,)), ((), ())),
                               preferred_element_type=jnp.float32)
             + conv_b_ref[...])
        h = jnp.maximum(h, 0.0)
        z = lax.dot_general(h, fc1_w_ref[...], (((1,), (0,)), ((), ())),
                            preferred_element_type=jnp.float32) + fc1_b_ref[...]
        z = jnp.where(z >= 0, z, prelu_ref[...] * z)
        logits = lax.dot_general(z, fc2_w_ref[...], (((1,), (0,)), ((), ())),
                                 preferred_element_type=jnp.float32) + fc2_b_ref[...]
        l0 = logits[:, 0:1]
        l1 = logits[:, 1:2]
        p0 = 1.0 / (1.0 + jnp.exp(l1 - l0))
        p1 = 1.0 / (1.0 + jnp.exp(l0 - l1))
        pred_ref[a] = jnp.concatenate([p0, p1], axis=1)[0:K, :]


def _gcn_tc(labs, idxs, sub_sum, features, conv_w, conv_b2, fc1_w, fc1_b2,
            prelu2, fc2_w, fc2_b2):
    grid = B // AB
    return pl.pallas_call(
        _gcn_body,
        grid=(grid,),
        in_specs=[
            pl.BlockSpec((1, AB, K), lambda i: (i, 0, 0), memory_space=pltpu.SMEM),
            pl.BlockSpec((1, 1, AB), lambda i: (i, 0, 0), memory_space=pltpu.SMEM),
            pl.BlockSpec(memory_space=pl.ANY),
            pl.BlockSpec(memory_space=pl.ANY),
            pl.BlockSpec((2 * D, NHID), lambda i: (0, 0)),
            pl.BlockSpec((1, NHID), lambda i: (0, 0)),
            pl.BlockSpec((NHID, NHID), lambda i: (0, 0)),
            pl.BlockSpec((1, NHID), lambda i: (0, 0)),
            pl.BlockSpec((1, NHID), lambda i: (0, 0)),
            pl.BlockSpec((NHID, NCLASS), lambda i: (0, 0)),
            pl.BlockSpec((1, NCLASS), lambda i: (0, 0)),
        ],
        out_specs=pl.BlockSpec((AB, K, NCLASS), lambda i: (i, 0, 0)),
        out_shape=jax.ShapeDtypeStruct((B, K, NCLASS), jnp.float32),
        scratch_shapes=[
            pltpu.VMEM((AB, KPAD, D), jnp.float32),
            pltpu.SemaphoreType.DMA,
        ],
    )(labs, idxs, sub_sum, features, conv_w, conv_b2, fc1_w, fc1_b2,
      prelu2, fc2_w, fc2_b2)


def kernel(indexes, features, labels, sub_label, domain, ori_0, ori_knn_neighbor,
           all_pred, output_feat, conv_w, conv_b, fc1_w, fc1_b, prelu_w, fc2_w, fc2_b):
    feat2 = features.reshape(N * NCHUNK, DC)
    lab3 = sub_label.reshape(NTILE, NBLK, BLK)
    zfeat = jnp.zeros((ACC_ROWS_PER_TILE, DC), jnp.float32)
    zcnt16 = jnp.zeros((ACC_ROWS_PER_TILE, 16), jnp.float32)
    sums3, cnt = _seg_sum_sc(feat2, lab3, zfeat, zcnt16)

    sub_sum, simm2, nums = _normalize_tc(sums3, cnt[:NSUB])

    labs = sub_label[ori_knn_neighbor]
    pred = _gcn_tc(labs.reshape(B // AB, AB, K), indexes.reshape(B // AB, 1, AB),
                   sub_sum, features,
                   conv_w, conv_b.reshape(1, NHID), fc1_w, fc1_b.reshape(1, NHID),
                   prelu_w.reshape(1, NHID), fc2_w, fc2_b.reshape(1, NCLASS))
    return pred, simm2.reshape(NSUB), sub_sum, nums


# speedup vs baseline: 1.2548x; 1.1591x over previous
import functools

import jax
import jax.numpy as jnp
from jax import lax
from jax.experimental import pallas as pl
from jax.experimental.pallas import tpu as pltpu
from jax.experimental.pallas import tpu_sc as plsc

N = 50000
D = 2048
NSUB = 5000
NSUB_PAD = 5120
B = 128
K = 30
KPAD = 32
NHID = 512
NCLASS = 2

DC = 256
NCHUNK = D // DC
NCORE = 2
NTILE = 16
ROWS_PER_TILE = N // NTILE
BLK = 125
NBLK = ROWS_PER_TILE // BLK
ACC_ROWS_PER_TILE = NSUB_PAD // NTILE


def _seg_sum_body(feat2, lab3, zfeat, zcnt16, out3, cnt,
                  acc, cacc, fbuf, labv, idxv, onesv):
    c = lax.axis_index("c")
    s = lax.axis_index("s")
    ov = jnp.ones((16,), jnp.float32)
    iv = lax.iota(jnp.int32, 16)

    @pl.loop(0, BLK)
    def _(i):
        onesv[i, :] = ov

    pltpu.sync_copy(lab3.at[s], labv)
    r0 = s * ACC_ROWS_PER_TILE
    row0 = s * ROWS_PER_TILE

    for p in range(NCHUNK // NCORE):
        dc = p * NCORE + c
        @pl.loop(0, NBLK)
        def _(b):
            for g in range(8):
                r = jnp.minimum(row0 + b * BLK + g * 16 + iv, N - 1)
                idxv[b, pl.ds(g * 16, 16)] = r * NCHUNK + dc
        pltpu.sync_copy(zfeat, acc.at[pl.ds(r0, ACC_ROWS_PER_TILE)])
        if p == 0:
            @pl.when(c == 0)
            def _():
                pltpu.sync_copy(zcnt16, cacc.at[pl.ds(r0, ACC_ROWS_PER_TILE)])
        plsc.subcore_barrier()
        for b in range(NBLK):
            pltpu.sync_copy(feat2.at[idxv.at[b]], fbuf)
            pltpu.sync_copy(fbuf.at[pl.ds(0, BLK)], acc.at[labv.at[b]], add=True)
            if p == 0:
                @pl.when(c == 0)
                def _():
                    pltpu.sync_copy(onesv, cacc.at[labv.at[b]], add=True)
        plsc.subcore_barrier()
        pltpu.sync_copy(acc.at[pl.ds(r0, ACC_ROWS_PER_TILE)],
                        out3.at[dc, pl.ds(r0, ACC_ROWS_PER_TILE)])
        if p == 0:
            @pl.when(c == 0)
            def _():
                pltpu.sync_copy(cacc.at[pl.ds(r0, ACC_ROWS_PER_TILE)],
                                cnt.at[pl.ds(r0, ACC_ROWS_PER_TILE)])
        plsc.subcore_barrier()


def _seg_sum_sc(feat2, lab3, zfeat, zcnt16):
    mesh = plsc.VectorSubcoreMesh(core_axis_name="c", subcore_axis_name="s")
    return pl.kernel(
        _seg_sum_body,
        out_type=(
            jax.ShapeDtypeStruct((NCHUNK, NSUB_PAD, DC), jnp.float32),
            jax.ShapeDtypeStruct((NSUB_PAD, 16), jnp.float32),
        ),
        mesh=mesh,
        compiler_params=pltpu.CompilerParams(use_tc_tiling_on_sc=False),
        scratch_types=[
            pltpu.VMEM_SHARED((NSUB_PAD, DC), jnp.float32),
            pltpu.VMEM_SHARED((NSUB_PAD, 16), jnp.float32),
            pltpu.VMEM((128, DC), jnp.float32),
            pltpu.VMEM((NBLK, BLK), jnp.int32),
            pltpu.VMEM((NBLK, 128), jnp.int32),
            pltpu.VMEM((BLK, 16), jnp.float32),
        ],
    )(feat2, lab3, zfeat, zcnt16)


NORM_ROWS = 200


def _norm_body(sums_ref, cnt_ref, sub_ref, simm_ref, nums_ref):
    c = cnt_ref[:, :1]
    scale = jnp.where(c > 0, 1.0 / jnp.where(c > 0, c, 1.0), 1.0)
    for q in range(NCHUNK):
        sub_ref[:, q * DC:(q + 1) * DC] = sums_ref[q] * scale
    w = sub_ref[...]
    simm_ref[...] = jnp.sum(w * w, axis=1, keepdims=True)
    nums_ref[...] = c


def _normalize_tc(sums3, cnt):
    grid = NSUB // NORM_ROWS
    return pl.pallas_call(
        _norm_body,
        grid=(grid,),
        in_specs=[
            pl.BlockSpec((NCHUNK, NORM_ROWS, DC), lambda i: (0, i, 0)),
            pl.BlockSpec((NORM_ROWS, 16), lambda i: (i, 0)),
        ],
        out_specs=[
            pl.BlockSpec((NORM_ROWS, D), lambda i: (i, 0)),
            pl.BlockSpec((NORM_ROWS, 1), lambda i: (i, 0)),
            pl.BlockSpec((NORM_ROWS, 1), lambda i: (i, 0)),
        ],
        out_shape=[
            jax.ShapeDtypeStruct((NSUB, D), jnp.float32),
            jax.ShapeDtypeStruct((NSUB, 1), jnp.float32),
            jax.ShapeDtypeStruct((NSUB, 1), jnp.float32),
        ],
    )(sums3, cnt)


AB = 8
NEG = -1e30


def _gcn_body(labs_ref, idxs_ref, sub_hbm, feat_hbm,
              conv_w_ref, conv_b_ref, fc1_w_ref, fc1_b_ref,
              prelu_ref, fc2_w_ref, fc2_b_ref,
              pred_ref, xbuf, cat_ref, sem):
    copies = []
    for a in range(AB):
        copies.append(pltpu.make_async_copy(
            feat_hbm.at[pl.ds(idxs_ref[0, 0, a], 1)], xbuf.at[a, pl.ds(0, 1)], sem))
        copies[-1].start()
        for j in range(1, K):
            copies.append(pltpu.make_async_copy(
                sub_hbm.at[pl.ds(labs_ref[0, a, j], 1)], xbuf.at[a, pl.ds(j, 1)], sem))
            copies[-1].start()
    for cp in copies:
        cp.wait()

    rows = lax.broadcasted_iota(jnp.int32, (KPAD, KPAD), 0)
    cols = lax.broadcasted_iota(jnp.int32, (KPAD, KPAD), 1)
    colpad = cols >= K
    eye = (rows == cols).astype(jnp.float32)

    for a in range(AB):
        xbuf[a, pl.ds(K, KPAD - K)] = jnp.zeros((KPAD - K, D), jnp.float32)
        X = xbuf[a]
        A = lax.dot_general(X, X, (((1,), (1,)), ((), ())),
                            preferred_element_type=jnp.float32) * 0.2
        cur = jnp.where(colpad, NEG, A)
        msk = jnp.zeros((KPAD, KPAD), jnp.float32)
        for _ in range(5):
            mx = jnp.max(cur, axis=1, keepdims=True)
            cand = jnp.where(cur == mx, cols, KPAD)
            cmin = jnp.min(cand, axis=1, keepdims=True)
            sel = cols == cmin
            msk = jnp.where(sel, 1.0, msk)
            cur = jnp.where(sel, NEG, cur)
        mskT = lax.dot_general(msk, eye, (((0,), (0,)), ((), ())),
                               preferred_element_type=jnp.float32)
        Am = A * msk * jnp.where(mskT > 0, 1.0, 0.0)
        n2 = jnp.sum(X * X, axis=1, keepdims=True)
        rn = jnp.where(n2 > 0, lax.rsqrt(jnp.where(n2 > 0, n2, 1.0)), 0.0)
        Xn = X * rn
        Xn = Xn - Xn[0:1, :]
        agg = lax.dot_general(Am, Xn, (((1,), (0,)), ((), ())),
                              preferred_element_type=jnp.float32)
        cat_ref[pl.ds(a * KPAD, KPAD), pl.ds(0, D)] = Xn
        cat_ref[pl.ds(a * KPAD, KPAD), pl.ds(D, D)] = agg

    cat = cat_ref[...]
    h = lax.dot_general(cat, conv_w_ref[...], (((1,), (0,)), ((), ())),
                        preferred_element_type=jnp.float32) + conv_b_ref[...]
    h = jnp.maximum(h, 0.0)
    z = lax.dot_general(h, fc1_w_ref[...], (((1,), (0,)), ((), ())),
                        preferred_element_type=jnp.float32) + fc1_b_ref[...]
    z = jnp.where(z >= 0, z, prelu_ref[...] * z)
    logits = lax.dot_general(z, fc2_w_ref[...], (((1,), (0,)), ((), ())),
                             preferred_element_type=jnp.float32) + fc2_b_ref[...]
    l0 = logits[:, 0:1]
    l1 = logits[:, 1:2]
    p0 = 1.0 / (1.0 + jnp.exp(l1 - l0))
    p1 = 1.0 / (1.0 + jnp.exp(l0 - l1))
    p01 = jnp.concatenate([p0, p1], axis=1)
    for a in range(AB):
        pred_ref[a] = p01[a * KPAD:a * KPAD + K, :]


def _gcn_tc(labs, idxs, sub_sum, features, conv_w, conv_b2, fc1_w, fc1_b2,
            prelu2, fc2_w, fc2_b2):
    grid = B // AB
    return pl.pallas_call(
        _gcn_body,
        grid=(grid,),
        in_specs=[
            pl.BlockSpec((1, AB, K), lambda i: (i, 0, 0), memory_space=pltpu.SMEM),
            pl.BlockSpec((1, 1, AB), lambda i: (i, 0, 0), memory_space=pltpu.SMEM),
            pl.BlockSpec(memory_space=pl.ANY),
            pl.BlockSpec(memory_space=pl.ANY),
            pl.BlockSpec((2 * D, NHID), lambda i: (0, 0)),
            pl.BlockSpec((1, NHID), lambda i: (0, 0)),
            pl.BlockSpec((NHID, NHID), lambda i: (0, 0)),
            pl.BlockSpec((1, NHID), lambda i: (0, 0)),
            pl.BlockSpec((1, NHID), lambda i: (0, 0)),
            pl.BlockSpec((NHID, NCLASS), lambda i: (0, 0)),
            pl.BlockSpec((1, NCLASS), lambda i: (0, 0)),
        ],
        out_specs=pl.BlockSpec((AB, K, NCLASS), lambda i: (i, 0, 0)),
        out_shape=jax.ShapeDtypeStruct((B, K, NCLASS), jnp.float32),
        scratch_shapes=[
            pltpu.VMEM((AB, KPAD, D), jnp.float32),
            pltpu.VMEM((AB * KPAD, 2 * D), jnp.float32),
            pltpu.SemaphoreType.DMA,
        ],
    )(labs, idxs, sub_sum, features, conv_w, conv_b2, fc1_w, fc1_b2,
      prelu2, fc2_w, fc2_b2)


def kernel(indexes, features, labels, sub_label, domain, ori_0, ori_knn_neighbor,
           all_pred, output_feat, conv_w, conv_b, fc1_w, fc1_b, prelu_w, fc2_w, fc2_b):
    feat2 = features.reshape(N * NCHUNK, DC)
    lab3 = sub_label.reshape(NTILE, NBLK, BLK)
    zfeat = jnp.zeros((ACC_ROWS_PER_TILE, DC), jnp.float32)
    zcnt16 = jnp.zeros((ACC_ROWS_PER_TILE, 16), jnp.float32)
    sums3, cnt = _seg_sum_sc(feat2, lab3, zfeat, zcnt16)

    sub_sum, simm2, nums = _normalize_tc(sums3, cnt[:NSUB])

    labs = sub_label[ori_knn_neighbor]
    pred = _gcn_tc(labs.reshape(B // AB, AB, K), indexes.reshape(B // AB, 1, AB),
                   sub_sum, features,
                   conv_w, conv_b.reshape(1, NHID), fc1_w, fc1_b.reshape(1, NHID),
                   prelu_w.reshape(1, NHID), fc2_w, fc2_b.reshape(1, NCLASS))
    return pred, simm2.reshape(NSUB), sub_sum, nums
